# baseline (speedup 1.0000x reference)
import functools

import jax
import jax.numpy as jnp
from jax import lax
from jax.experimental import pallas as pl
from jax.experimental.pallas import tpu as pltpu
from jax.experimental.pallas import tpu_sc as plsc

T, D, E, F = 2048, 1024, 8, 4096
TB = 512
NB = 16
FB = 512
NBP = 24

_INTERPRET = False


def _router_kernel(x_ref, rw_ref, logits_ref, meta_f_ref, meta_i_ref, bexp_ref):
    f32 = jnp.float32
    x = x_ref[...]
    rw = rw_ref[...]
    logits = lax.dot_general(x, rw, (((1,), (1,)), ((), ())),
                             preferred_element_type=f32)
    logits_ref[...] = logits
    m = jnp.max(logits, axis=1, keepdims=True)
    ex = jnp.exp(logits - m)
    p = ex / jnp.sum(ex, axis=1, keepdims=True)
    lane_e = lax.broadcasted_iota(jnp.int32, (T, E), 1)
    m1 = jnp.max(p, axis=1, keepdims=True)
    i1 = jnp.min(jnp.where(p == m1, lane_e, E), axis=1, keepdims=True)
    pm = jnp.where(lane_e == i1, -1e30, p)
    m2 = jnp.max(pm, axis=1, keepdims=True)
    i2 = jnp.min(jnp.where(pm == m2, lane_e, E), axis=1, keepdims=True)
    denom = m1 + m2
    w0 = m1 / denom
    w1 = m2 / denom
    oh1 = (lane_e == i1).astype(f32)
    oh2 = (lane_e == i2).astype(f32)
    S = oh1 + oh2

    G, R = 16, 128
    ri = lax.broadcasted_iota(jnp.int32, (R, R), 0)
    ci = lax.broadcasted_iota(jnp.int32, (R, R), 1)
    L128 = (ri > ci).astype(f32)
    Cs = []
    Tg = []
    for g in range(G):
        Sg = lax.slice(S, (g * R, 0), ((g + 1) * R, E))
        Cs.append(lax.dot_general(L128, Sg, (((1,), (0,)), ((), ())),
                                  preferred_element_type=f32))
        Tg.append(jnp.sum(Sg, axis=0, keepdims=True))
    Tmat = jnp.concatenate(Tg, axis=0)
    gri = lax.broadcasted_iota(jnp.int32, (G, G), 0)
    gci = lax.broadcasted_iota(jnp.int32, (G, G), 1)
    LG = (gri > gci).astype(f32)
    GT = lax.dot_general(LG, Tmat, (((1,), (0,)), ((), ())),
                         preferred_element_type=f32)
    P = jnp.concatenate(
        [Cs[g] + lax.slice(GT, (g, 0), (g + 1, E)) for g in range(G)], axis=0)
    R0 = jnp.sum(oh1 * P, axis=1, keepdims=True)
    R1 = jnp.sum(oh2 * (P + oh1), axis=1, keepdims=True)
    cnt = jnp.sum(S, axis=0, keepdims=True)
    pc = jnp.ceil(cnt / TB) * TB
    eri = lax.broadcasted_iota(jnp.int32, (E, E), 0)
    eci = lax.broadcasted_iota(jnp.int32, (E, E), 1)
    U = (eri < eci).astype(f32)
    Off = lax.dot_general(pc, U, (((1,), (0,)), ((), ())),
                          preferred_element_type=f32)
    dest0 = jnp.sum(oh1 * Off, axis=1, keepdims=True) + R0
    dest1 = jnp.sum(oh2 * Off, axis=1, keepdims=True) + R1
    zpad = jnp.zeros((T, E - 2), f32)
    meta_f_ref[...] = jnp.concatenate([w0, w1, zpad], axis=1)
    meta_i_ref[...] = jnp.concatenate([dest0, dest1, zpad], axis=1).astype(jnp.int32)
    boff = Off * (1.0 / TB)
    bi = lax.broadcasted_iota(jnp.int32, (NBP, E), 0).astype(f32)
    bexp = jnp.sum((bi >= boff).astype(f32), axis=1, keepdims=True) - 1.0
    nact = jnp.sum(pc, axis=1, keepdims=True) * (1.0 / TB)
    bexp = jnp.where(bi[:, :1] == NB, nact, bexp)
    bexp_ref[...] = jnp.broadcast_to(bexp, (NBP, E)).astype(jnp.int32)


def _router_call(x, rw):
    return pl.pallas_call(
        _router_kernel,
        out_shape=[
            jax.ShapeDtypeStruct((T, E), jnp.float32),
            jax.ShapeDtypeStruct((T, E), jnp.float32),
            jax.ShapeDtypeStruct((T, E), jnp.int32),
            jax.ShapeDtypeStruct((NBP, E), jnp.int32),
        ],
        interpret=_INTERPRET,
    )(x, rw)


def _gelu_exact(h):
    return 0.5 * h * (1.0 + lax.erf(h * (2.0 ** -0.5)))


NFC = F // FB


def _expert_kernel(s_ref, xg_ref, w1_ref, v_ref, d_ref, out_ref):
    f32 = jnp.float32
    bf16 = jnp.bfloat16
    fc = pl.program_id(0)
    nb = pl.program_id(1)
    nact = s_ref[NB]

    @pl.when(nb < nact)
    def _():
        xb = xg_ref[...]
        w1c = w1_ref[0].astype(bf16)
        vc = v_ref[0].astype(bf16)
        dc = d_ref[0].astype(bf16)
        h1 = lax.dot_general(xb, w1c, (((1,), (1,)), ((), ())),
                             preferred_element_type=f32)
        hv = lax.dot_general(xb, vc, (((1,), (1,)), ((), ())),
                             preferred_element_type=f32)
        h = (_gelu_exact(h1) * hv).astype(bf16)
        part = lax.dot_general(h, dc, (((1,), (1,)), ((), ())),
                               preferred_element_type=f32)
        sl = pl.ds(nb * TB, TB)

        @pl.when(fc == 0)
        def _():
            out_ref[sl, :] = part

        @pl.when(fc != 0)
        def _():
            out_ref[sl, :] += part


def _expert_call(s, xg, w1, v, dense):
    grid_spec = pltpu.PrefetchScalarGridSpec(
        num_scalar_prefetch=1,
        grid=(NFC, NB),
        in_specs=[
            pl.BlockSpec((TB, D),
                         lambda fc, nb, s: (jnp.minimum(nb, s[NB] - 1), 0)),
            pl.BlockSpec((1, FB, D), lambda fc, nb, s: (s[nb], fc, 0)),
            pl.BlockSpec((1, FB, D), lambda fc, nb, s: (s[nb], fc, 0)),
            pl.BlockSpec((1, D, FB), lambda fc, nb, s: (s[nb], 0, fc)),
        ],
        out_specs=pl.BlockSpec((NB * TB, D), lambda fc, nb, s: (0, 0)),
    )
    return pl.pallas_call(
        _expert_kernel,
        grid_spec=grid_spec,
        out_shape=jax.ShapeDtypeStruct((NB * TB, D), jnp.float32),
        interpret=_INTERPRET,
    )(s, xg, w1, v, dense)


NW = 32
RW = T // NW
CH = 32

_SC_MESH = plsc.VectorSubcoreMesh(core_axis_name="c", subcore_axis_name="s")


@functools.partial(
    pl.kernel,
    out_type=jax.ShapeDtypeStruct((NB * TB, D // 2), jnp.int32),
    mesh=_SC_MESH,
    scratch_types=[
        pltpu.VMEM((RW, D // 2), jnp.int32),
        pltpu.VMEM((RW,), jnp.int32),
        pltpu.VMEM((RW,), jnp.int32),
        pltpu.SemaphoreType.DMA,
    ],
)
def _sc_scatter(x_hbm, d0_hbm, d1_hbm, xg_hbm, rows_v, i0_v, i1_v, sem):
    wid = lax.axis_index("s") * 2 + lax.axis_index("c")
    base = wid * RW
    pltpu.sync_copy(d0_hbm.at[pl.ds(base, RW)], i0_v)
    pltpu.sync_copy(d1_hbm.at[pl.ds(base, RW)], i1_v)
    pltpu.sync_copy(x_hbm.at[pl.ds(base, RW)], rows_v)
    pltpu.async_copy(rows_v, xg_hbm.at[i0_v], sem).wait()
    pltpu.async_copy(rows_v, xg_hbm.at[i1_v], sem).wait()


@functools.partial(
    pl.kernel,
    out_type=jax.ShapeDtypeStruct((T, D), jnp.float32),
    mesh=_SC_MESH,
    scratch_types=[
        pltpu.VMEM((CH, D), jnp.float32),
        pltpu.VMEM((CH, D), jnp.float32),
        pltpu.VMEM((RW,), jnp.int32),
        pltpu.VMEM((RW,), jnp.int32),
        pltpu.VMEM((RW + 16,), jnp.float32),
        pltpu.VMEM((RW + 16,), jnp.float32),
        pltpu.SemaphoreType.DMA,
    ],
)
def _sc_combine(contrib_hbm, d0_hbm, d1_hbm, w0_hbm, w1_hbm, out_hbm,
                g_v, acc_v, i0_v, i1_v, w0_v, w1_v, sem):
    wid = lax.axis_index("s") * 2 + lax.axis_index("c")
    base = wid * RW
    pltpu.sync_copy(d0_hbm.at[pl.ds(base, RW)], i0_v)
    pltpu.sync_copy(d1_hbm.at[pl.ds(base, RW)], i1_v)
    pltpu.sync_copy(w0_hbm.at[pl.ds(base, RW)], w0_v.at[pl.ds(0, RW)])
    pltpu.sync_copy(w1_hbm.at[pl.ds(base, RW)], w1_v.at[pl.ds(0, RW)])
    for ch in range(RW // CH):
        pltpu.async_copy(contrib_hbm.at[i0_v.at[pl.ds(ch * CH, CH)]],
                         g_v, sem).wait()

        def mul_row(r, _):
            w = w0_v[pl.ds(ch * CH + r, 16)][0]
            for c in range(D // 16):
                acc_v[r, pl.ds(c * 16, 16)] = g_v[r, pl.ds(c * 16, 16)] * w
            return 0

        lax.fori_loop(0, CH, mul_row, 0)
        pltpu.async_copy(contrib_hbm.at[i1_v.at[pl.ds(ch * CH, CH)]],
                         g_v, sem).wait()

        def fma_row(r, _):
            w = w1_v[pl.ds(ch * CH + r, 16)][0]
            for c in range(D // 16):
                acc_v[r, pl.ds(c * 16, 16)] += g_v[r, pl.ds(c * 16, 16)] * w
            return 0

        lax.fori_loop(0, CH, fma_row, 0)
        pltpu.sync_copy(acc_v, out_hbm.at[pl.ds(base + ch * CH, CH)])


def kernel(hidden_states, router_w, w1, v, dense):
    b, s, d = hidden_states.shape
    x = hidden_states.reshape(b * s, d)
    logits, meta_f, meta_i, bexp_pad = _router_call(x, router_w)
    wgt0 = meta_f[:, 0]
    wgt1 = meta_f[:, 1]
    dest0 = meta_i[:, 0]
    dest1 = meta_i[:, 1]
    scal = bexp_pad[:NB + 1, 0]
    xb16 = x.astype(jnp.bfloat16).reshape(T, D // 2, 2)
    xi32 = lax.bitcast_convert_type(xb16, jnp.int32)
    xgi = _sc_scatter(xi32, dest0, dest1)
    xg = lax.bitcast_convert_type(xgi, jnp.bfloat16).reshape(NB * TB, D)
    contrib = _expert_call(scal, xg, w1, v, dense)
    out = _sc_combine(contrib, dest0, dest1, wgt0, wgt1)
    return out.reshape(b, s, d), logits


# speedup vs baseline: 1.4427x; 1.4427x over previous
import functools

import jax
import jax.numpy as jnp
from jax import lax
from jax.experimental import pallas as pl
from jax.experimental.pallas import tpu as pltpu
from jax.experimental.pallas import tpu_sc as plsc

T, D, E, F = 2048, 1024, 8, 4096
TB = 512
NB = 16
FB = 512
NBP = 24

_INTERPRET = False


def _router_kernel(x_ref, rw_ref, logits_ref, meta_f_ref, meta_i_ref, bexp_ref):
    f32 = jnp.float32
    x = x_ref[...]
    rw = rw_ref[...]
    logits = lax.dot_general(x, rw, (((1,), (1,)), ((), ())),
                             preferred_element_type=f32)
    logits_ref[...] = logits
    m = jnp.max(logits, axis=1, keepdims=True)
    ex = jnp.exp(logits - m)
    p = ex / jnp.sum(ex, axis=1, keepdims=True)
    lane_e = lax.broadcasted_iota(jnp.int32, (T, E), 1)
    m1 = jnp.max(p, axis=1, keepdims=True)
    i1 = jnp.min(jnp.where(p == m1, lane_e, E), axis=1, keepdims=True)
    pm = jnp.where(lane_e == i1, -1e30, p)
    m2 = jnp.max(pm, axis=1, keepdims=True)
    i2 = jnp.min(jnp.where(pm == m2, lane_e, E), axis=1, keepdims=True)
    denom = m1 + m2
    w0 = m1 / denom
    w1 = m2 / denom
    oh1 = (lane_e == i1).astype(f32)
    oh2 = (lane_e == i2).astype(f32)
    S = oh1 + oh2

    G, R = 16, 128
    ri = lax.broadcasted_iota(jnp.int32, (R, R), 0)
    ci = lax.broadcasted_iota(jnp.int32, (R, R), 1)
    L128 = (ri > ci).astype(f32)
    Cs = []
    Tg = []
    for g in range(G):
        Sg = lax.slice(S, (g * R, 0), ((g + 1) * R, E))
        Cs.append(lax.dot_general(L128, Sg, (((1,), (0,)), ((), ())),
                                  preferred_element_type=f32))
        Tg.append(jnp.sum(Sg, axis=0, keepdims=True))
    Tmat = jnp.concatenate(Tg, axis=0)
    gri = lax.broadcasted_iota(jnp.int32, (G, G), 0)
    gci = lax.broadcasted_iota(jnp.int32, (G, G), 1)
    LG = (gri > gci).astype(f32)
    GT = lax.dot_general(LG, Tmat, (((1,), (0,)), ((), ())),
                         preferred_element_type=f32)
    P = jnp.concatenate(
        [Cs[g] + lax.slice(GT, (g, 0), (g + 1, E)) for g in range(G)], axis=0)
    R0 = jnp.sum(oh1 * P, axis=1, keepdims=True)
    R1 = jnp.sum(oh2 * (P + oh1), axis=1, keepdims=True)
    cnt = jnp.sum(S, axis=0, keepdims=True)
    pc = jnp.ceil(cnt / TB) * TB
    eri = lax.broadcasted_iota(jnp.int32, (E, E), 0)
    eci = lax.broadcasted_iota(jnp.int32, (E, E), 1)
    U = (eri < eci).astype(f32)
    Off = lax.dot_general(pc, U, (((1,), (0,)), ((), ())),
                          preferred_element_type=f32)
    dest0 = jnp.sum(oh1 * Off, axis=1, keepdims=True) + R0
    dest1 = jnp.sum(oh2 * Off, axis=1, keepdims=True) + R1
    zpad = jnp.zeros((T, E - 2), f32)
    meta_f_ref[...] = jnp.concatenate([w0, w1, zpad], axis=1)
    meta_i_ref[...] = jnp.concatenate([dest0, dest1, zpad], axis=1).astype(jnp.int32)
    boff = Off * (1.0 / TB)
    bi = lax.broadcasted_iota(jnp.int32, (NBP, E), 0).astype(f32)
    bexp = jnp.sum((bi >= boff).astype(f32), axis=1, keepdims=True) - 1.0
    nact = jnp.sum(pc, axis=1, keepdims=True) * (1.0 / TB)
    bexp = jnp.where(bi[:, :1] == NB, nact, bexp)
    bexp_ref[...] = jnp.broadcast_to(bexp, (NBP, E)).astype(jnp.int32)


def _router_call(x, rw):
    return pl.pallas_call(
        _router_kernel,
        out_shape=[
            jax.ShapeDtypeStruct((T, E), jnp.float32),
            jax.ShapeDtypeStruct((T, E), jnp.float32),
            jax.ShapeDtypeStruct((T, E), jnp.int32),
            jax.ShapeDtypeStruct((NBP, E), jnp.int32),
        ],
        interpret=_INTERPRET,
    )(x, rw)


def _gelu_exact(h):
    return 0.5 * h * (1.0 + lax.erf(h * (2.0 ** -0.5)))


NFC = F // FB


def _expert_kernel(s_ref, xg_ref, w1_ref, v_ref, d_ref, out_ref):
    f32 = jnp.float32
    bf16 = jnp.bfloat16
    fc = pl.program_id(0)
    nb = pl.program_id(1)
    nact = s_ref[NB]

    @pl.when(nb < nact)
    def _():
        xb = xg_ref[...]
        w1c = w1_ref[0].astype(bf16)
        vc = v_ref[0].astype(bf16)
        dc = d_ref[0].astype(bf16)
        h1 = lax.dot_general(xb, w1c, (((1,), (1,)), ((), ())),
                             preferred_element_type=f32)
        hv = lax.dot_general(xb, vc, (((1,), (1,)), ((), ())),
                             preferred_element_type=f32)
        h = (_gelu_exact(h1) * hv).astype(bf16)
        part = lax.dot_general(h, dc, (((1,), (1,)), ((), ())),
                               preferred_element_type=f32)
        sl = pl.ds(nb * TB, TB)

        @pl.when(fc == 0)
        def _():
            out_ref[sl, :] = part

        @pl.when(fc != 0)
        def _():
            out_ref[sl, :] += part


def _expert_call(s, xg, w1, v, dense):
    grid_spec = pltpu.PrefetchScalarGridSpec(
        num_scalar_prefetch=1,
        grid=(NFC, NB),
        in_specs=[
            pl.BlockSpec((TB, D),
                         lambda fc, nb, s: (jnp.minimum(nb, s[NB] - 1), 0)),
            pl.BlockSpec((1, FB, D), lambda fc, nb, s: (s[nb], fc, 0)),
            pl.BlockSpec((1, FB, D), lambda fc, nb, s: (s[nb], fc, 0)),
            pl.BlockSpec((1, D, FB), lambda fc, nb, s: (s[nb], 0, fc)),
        ],
        out_specs=pl.BlockSpec((NB * TB, D), lambda fc, nb, s: (0, 0)),
    )
    return pl.pallas_call(
        _expert_kernel,
        grid_spec=grid_spec,
        out_shape=jax.ShapeDtypeStruct((NB * TB, D), jnp.float32),
        interpret=_INTERPRET,
    )(s, xg, w1, v, dense)


NW = 32
RW = T // NW
CH = 32

_SC_MESH = plsc.VectorSubcoreMesh(core_axis_name="c", subcore_axis_name="s")


@functools.partial(
    pl.kernel,
    out_type=jax.ShapeDtypeStruct((NB * TB, D), jnp.float32),
    mesh=_SC_MESH,
    scratch_types=[
        pltpu.VMEM((RW, D), jnp.float32),
        pltpu.VMEM((RW,), jnp.int32),
        pltpu.VMEM((RW,), jnp.int32),
        pltpu.SemaphoreType.DMA,
    ],
)
def _sc_scatter(x_hbm, d0_hbm, d1_hbm, xg_hbm, rows_v, i0_v, i1_v, sem):
    wid = lax.axis_index("s") * 2 + lax.axis_index("c")
    base = wid * RW
    pltpu.sync_copy(d0_hbm.at[pl.ds(base, RW)], i0_v)
    pltpu.sync_copy(d1_hbm.at[pl.ds(base, RW)], i1_v)
    pltpu.sync_copy(x_hbm.at[pl.ds(base, RW)], rows_v)
    pltpu.async_copy(rows_v, xg_hbm.at[i0_v], sem).wait()
    pltpu.async_copy(rows_v, xg_hbm.at[i1_v], sem).wait()


@functools.partial(
    pl.kernel,
    out_type=jax.ShapeDtypeStruct((T, D), jnp.float32),
    mesh=_SC_MESH,
    scratch_types=[
        pltpu.VMEM((CH, D), jnp.float32),
        pltpu.VMEM((CH, D), jnp.float32),
        pltpu.VMEM((RW,), jnp.int32),
        pltpu.VMEM((RW,), jnp.int32),
        pltpu.VMEM((RW + 16,), jnp.float32),
        pltpu.VMEM((RW + 16,), jnp.float32),
        pltpu.SemaphoreType.DMA,
    ],
)
def _sc_combine(contrib_hbm, d0_hbm, d1_hbm, w0_hbm, w1_hbm, out_hbm,
                g_v, acc_v, i0_v, i1_v, w0_v, w1_v, sem):
    wid = lax.axis_index("s") * 2 + lax.axis_index("c")
    base = wid * RW
    pltpu.sync_copy(d0_hbm.at[pl.ds(base, RW)], i0_v)
    pltpu.sync_copy(d1_hbm.at[pl.ds(base, RW)], i1_v)
    pltpu.sync_copy(w0_hbm.at[pl.ds(base, RW)], w0_v.at[pl.ds(0, RW)])
    pltpu.sync_copy(w1_hbm.at[pl.ds(base, RW)], w1_v.at[pl.ds(0, RW)])
    for ch in range(RW // CH):
        pltpu.async_copy(contrib_hbm.at[i0_v.at[pl.ds(ch * CH, CH)]],
                         g_v, sem).wait()

        def mul_row(r, _):
            w = w0_v[pl.ds(ch * CH + r, 16)][0]
            for c in range(D // 16):
                acc_v[r, pl.ds(c * 16, 16)] = g_v[r, pl.ds(c * 16, 16)] * w
            return 0

        lax.fori_loop(0, CH, mul_row, 0)
        pltpu.async_copy(contrib_hbm.at[i1_v.at[pl.ds(ch * CH, CH)]],
                         g_v, sem).wait()

        def fma_row(r, _):
            w = w1_v[pl.ds(ch * CH + r, 16)][0]
            for c in range(D // 16):
                acc_v[r, pl.ds(c * 16, 16)] += g_v[r, pl.ds(c * 16, 16)] * w
            return 0

        lax.fori_loop(0, CH, fma_row, 0)
        pltpu.sync_copy(acc_v, out_hbm.at[pl.ds(base + ch * CH, CH)])


def kernel(hidden_states, router_w, w1, v, dense):
    b, s, d = hidden_states.shape
    x = hidden_states.reshape(b * s, d)
    logits, meta_f, meta_i, bexp_pad = _router_call(x, router_w)
    wgt0 = meta_f[:, 0]
    wgt1 = meta_f[:, 1]
    dest0 = meta_i[:, 0]
    dest1 = meta_i[:, 1]
    scal = bexp_pad[:NB + 1, 0]
    xg = _sc_scatter(x, dest0, dest1)
    contrib = _expert_call(scal, xg.astype(jnp.bfloat16), w1, v, dense)
    out = _sc_combine(contrib, dest0, dest1, wgt0, wgt1)
    return out.reshape(b, s, d), logits
